# Initial kernel scaffold; baseline (speedup 1.0000x reference)
import jax
import jax.numpy as jnp
from jax.experimental import pallas as pl


def kernel(input, target):
    raise NotImplementedError("write your pallas kernel here")


# speedup vs baseline: 3.7992x; 3.7992x over previous
import jax
import jax.numpy as jnp
from jax.experimental import pallas as pl
from jax.experimental.pallas import tpu as pltpu

_N = 8192
_C = 256
_BLOCK_ROWS = 512
_GRID = _N // _BLOCK_ROWS


def _loss_kernel(x_ref, t_ref, out_ref, c_ref, s_ref):
    i = pl.program_id(0)

    @pl.when(i == 0)
    def _init():
        t = t_ref[...]
        ks = jax.lax.broadcasted_iota(jnp.int32, (_C, _N), 0)
        onehot = (t == ks).astype(jnp.float32)
        counts = jnp.sum(onehot, axis=1, keepdims=True)
        recip = 1.0 / jnp.maximum(counts, 1.0)
        wvec = jnp.sum(onehot * recip, axis=0, keepdims=True)
        c = t.astype(jnp.float32) * wvec
        c_ref[...] = c
        s_ref[0] = jnp.sum(c)
        out_ref[...] = jnp.zeros_like(out_ref)

    x = x_ref[...]
    m = jnp.max(x, axis=1, keepdims=True)
    lse = jnp.log(jnp.sum(jnp.exp(x - m), axis=1, keepdims=True)) + m
    lse_sum = jnp.sum(lse)
    colsum = jnp.sum(x, axis=0, keepdims=True)
    wdot = jnp.sum(colsum * c_ref[...])
    val = (s_ref[0] * lse_sum - wdot) * (1.0 / _N)
    out_ref[...] += jnp.full((1, 1), 1.0, jnp.float32) * val


def kernel(input, target):
    t2d = target.reshape(1, _N)
    out = pl.pallas_call(
        _loss_kernel,
        grid=(_GRID,),
        in_specs=[
            pl.BlockSpec((_BLOCK_ROWS, _N), lambda i: (i, 0)),
            pl.BlockSpec((1, _N), lambda i: (0, 0)),
        ],
        out_specs=pl.BlockSpec((1, 1), lambda i: (0, 0)),
        out_shape=jax.ShapeDtypeStruct((1, 1), jnp.float32),
        scratch_shapes=[
            pltpu.VMEM((1, _N), jnp.float32),
            pltpu.SMEM((1,), jnp.float32),
        ],
    )(input, t2d)
    return out.reshape(())
